# baseline (speedup 1.0000x reference)
import functools

import jax
import jax.numpy as jnp
from jax import lax
from jax.experimental import pallas as pl
from jax.experimental.pallas import tpu as pltpu
from jax.experimental.pallas import tpu_sc as plsc

NUM_ROWS = 1000000
B = 4096
L = 50
D = 64
O = 128
NC, NS = 2, 16
NW = NC * NS
BPW = B // NW


def _pool_body(vt_hbm, table_hbm, out_hbm, raw_v, idx_v, acc_v, sem, isem):
    wid = lax.axis_index("s") * NC + lax.axis_index("c")
    base = wid * BPW
    idx_cp = pltpu.async_copy(vt_hbm.at[pl.ds(base, BPW)], raw_v, isem)

    zeros = jnp.zeros((16,), jnp.float32)

    def zbody(i, carry):
        for g in range(D // 16):
            acc_v[i, pl.ds(g * 16, 16)] = zeros
        return carry

    lax.fori_loop(0, BPW, zbody, 0)
    idx_cp.wait()

    lanes = lax.iota(jnp.int32, 16)

    def tbody(j, carry):
        cols = jnp.full((16,), j, jnp.int32)
        for g in range(BPW // 16):
            v = plsc.load_gather(raw_v, [lanes + g * 16, cols])
            idx_v[j, pl.ds(g * 16, 16)] = v + v
        return carry

    lax.fori_loop(0, L, tbody, 0)

    def fire(j, carry):
        pltpu.async_copy(table_hbm.at[idx_v.at[j]], acc_v, sem, add=True)
        return carry

    lax.fori_loop(0, L, fire, 0)

    def drain(j, carry):
        pltpu.make_async_copy(table_hbm.at[idx_v.at[0]], acc_v, sem).wait()
        return carry

    lax.fori_loop(0, L, drain, 0)
    pltpu.sync_copy(acc_v, out_hbm.at[pl.ds(base, BPW)])


_pool = functools.partial(
    pl.kernel,
    out_type=jax.ShapeDtypeStruct((B, D), jnp.float32),
    mesh=plsc.VectorSubcoreMesh(core_axis_name="c", subcore_axis_name="s"),
    scratch_types=[
        pltpu.VMEM((BPW, L), jnp.int32),
        pltpu.VMEM((L, BPW), jnp.int32),
        pltpu.VMEM((BPW, D), jnp.float32),
        pltpu.SemaphoreType.DMA,
        pltpu.SemaphoreType.DMA,
    ],
    compiler_params=pltpu.CompilerParams(
        use_tc_tiling_on_sc=False, needs_layout_passes=False
    ),
)(_pool_body)


RC = 16384


def _repack_body(t_ref, o_ref):
    y = jnp.transpose(t_ref[...])
    o_ref[:, 0:D] = y
    o_ref[:, D : 2 * D] = y


def _repack(t):
    grid = (NUM_ROWS + RC - 1) // RC
    return pl.pallas_call(
        _repack_body,
        out_shape=jax.ShapeDtypeStruct((NUM_ROWS, 2 * D), jnp.float32),
        grid=(grid,),
        in_specs=[pl.BlockSpec((D, RC), lambda i: (0, i))],
        out_specs=pl.BlockSpec((RC, 2 * D), lambda i: (i, 0)),
    )(t)


def _linear_body(x_ref, w_ref, b_ref, o_ref):
    x = x_ref[...] * jnp.float32(1.0 / L)
    o_ref[...] = (
        lax.dot_general(
            x, w_ref[...], (((1,), (1,)), ((), ())),
            preferred_element_type=jnp.float32,
        )
        + b_ref[...]
    )


def _linear(pooled, W, b2d):
    blk = 512
    return pl.pallas_call(
        _linear_body,
        out_shape=jax.ShapeDtypeStruct((B, O), jnp.float32),
        grid=(B // blk,),
        in_specs=[
            pl.BlockSpec((blk, D), lambda i: (i, 0)),
            pl.BlockSpec((O, D), lambda i: (0, 0)),
            pl.BlockSpec((1, O), lambda i: (0, 0)),
        ],
        out_specs=pl.BlockSpec((blk, O), lambda i: (i, 0)),
    )(pooled, W, b2d)


def kernel(vectorized_text, emb_table, W, b):
    packed = _repack(emb_table.T)
    pooled = _pool(
        vectorized_text.astype(jnp.int32),
        packed.reshape(2 * NUM_ROWS, D),
    )
    return _linear(pooled, W, b.reshape(1, O))


# speedup vs baseline: 1.1611x; 1.1611x over previous
import functools

import jax
import jax.numpy as jnp
from jax import lax
from jax.experimental import pallas as pl
from jax.experimental.pallas import tpu as pltpu
from jax.experimental.pallas import tpu_sc as plsc

NUM_ROWS = 1000000
B = 4096
L = 50
D = 64
O = 128
NC, NS = 2, 16
NW = NC * NS
BPW = B // NW


def _pool_body(vt_hbm, table_hbm, out_hbm, raw_v, idx_v, acc_v, sem, isem):
    wid = lax.axis_index("s") * NC + lax.axis_index("c")
    base = wid * BPW
    idx_cp = pltpu.async_copy(vt_hbm.at[pl.ds(base, BPW)], raw_v, isem)

    zeros = jnp.zeros((16,), jnp.float32)

    def zbody(i, carry):
        for g in range(D // 16):
            acc_v[i, pl.ds(g * 16, 16)] = zeros
        return carry

    lax.fori_loop(0, BPW, zbody, 0)
    idx_cp.wait()

    lanes = lax.iota(jnp.int32, 16)

    def tbody(j, carry):
        cols = jnp.full((16,), j, jnp.int32)
        for g in range(BPW // 16):
            v = plsc.load_gather(raw_v, [lanes + g * 16, cols])
            idx_v[j, pl.ds(g * 16, 16)] = v + v
        return carry

    lax.fori_loop(0, L, tbody, 0)

    def fire(j, carry):
        pltpu.async_copy(table_hbm.at[idx_v.at[j]], acc_v, sem, add=True)
        return carry

    lax.fori_loop(0, L, fire, 0)

    def drain(j, carry):
        pltpu.make_async_copy(table_hbm.at[idx_v.at[0]], acc_v, sem).wait()
        return carry

    lax.fori_loop(0, L, drain, 0)
    pltpu.sync_copy(acc_v, out_hbm.at[pl.ds(base, BPW)])


_pool = functools.partial(
    pl.kernel,
    out_type=jax.ShapeDtypeStruct((B, D), jnp.float32),
    mesh=plsc.VectorSubcoreMesh(core_axis_name="c", subcore_axis_name="s"),
    scratch_types=[
        pltpu.VMEM((BPW, L), jnp.int32),
        pltpu.VMEM((L, BPW), jnp.int32),
        pltpu.VMEM((BPW, D), jnp.float32),
        pltpu.SemaphoreType.DMA,
        pltpu.SemaphoreType.DMA,
    ],
    compiler_params=pltpu.CompilerParams(
        use_tc_tiling_on_sc=False, needs_layout_passes=False
    ),
)(_pool_body)


RC = 32768


def _repack_body(t_ref, o_ref):
    o_ref[:, 0:D] = jnp.transpose(t_ref[...])


def _repack(t):
    grid = (NUM_ROWS + RC - 1) // RC
    return pl.pallas_call(
        _repack_body,
        out_shape=jax.ShapeDtypeStruct((NUM_ROWS, 2 * D), jnp.float32),
        grid=(grid,),
        in_specs=[pl.BlockSpec((D, RC), lambda i: (0, i))],
        out_specs=pl.BlockSpec((RC, 2 * D), lambda i: (i, 0)),
    )(t)


def _linear_body(x_ref, w_ref, b_ref, o_ref):
    x = x_ref[...] * jnp.float32(1.0 / L)
    o_ref[...] = (
        lax.dot_general(
            x, w_ref[...], (((1,), (1,)), ((), ())),
            preferred_element_type=jnp.float32,
        )
        + b_ref[...]
    )


def _linear(pooled, W, b2d):
    blk = 512
    return pl.pallas_call(
        _linear_body,
        out_shape=jax.ShapeDtypeStruct((B, O), jnp.float32),
        grid=(B // blk,),
        in_specs=[
            pl.BlockSpec((blk, D), lambda i: (i, 0)),
            pl.BlockSpec((O, D), lambda i: (0, 0)),
            pl.BlockSpec((1, O), lambda i: (0, 0)),
        ],
        out_specs=pl.BlockSpec((blk, O), lambda i: (i, 0)),
    )(pooled, W, b2d)


def kernel(vectorized_text, emb_table, W, b):
    packed = _repack(emb_table.T)
    pooled = _pool(
        vectorized_text.astype(jnp.int32),
        packed.reshape(2 * NUM_ROWS, D),
    )
    return _linear(pooled, W, b.reshape(1, O))
